# baseline (speedup 1.0000x reference)
import functools
import jax
import jax.numpy as jnp
from jax import lax
from jax.experimental import pallas as pl
from jax.experimental.pallas import tpu as pltpu
from jax.experimental.pallas import tpu_sc as plsc

NC = 2
NS = 16
NW = NC * NS
CH = 8
SB = 2
D = 1024
NV = D // 16
NBUF = 3
BLOCK_S = 512


def _add_kernel(x_ref, w_ref, o_ref):
    o_ref[...] = x_ref[...] + w_ref[...]


def _tc_part(inputs, weight):
    batch, seq_len, dim = inputs.shape
    return pl.pallas_call(
        _add_kernel,
        grid=(seq_len // BLOCK_S,),
        in_specs=[
            pl.BlockSpec((SB, BLOCK_S, dim), lambda s: (1, s, 0)),
            pl.BlockSpec((BLOCK_S, dim), lambda s: (s, 0)),
        ],
        out_specs=pl.BlockSpec((SB, BLOCK_S, dim), lambda s: (0, s, 0)),
        out_shape=jax.ShapeDtypeStruct((SB, seq_len, dim), inputs.dtype),
    )(inputs, weight)


def _sc_part(inputs, weight):
    batch, seq_len, dim = inputs.shape
    per_w = seq_len // NW
    n_chunks = per_w // CH

    mesh = plsc.VectorSubcoreMesh(core_axis_name="c", subcore_axis_name="s")

    @functools.partial(
        pl.kernel,
        mesh=mesh,
        out_type=jax.ShapeDtypeStruct((SB, seq_len, dim), jnp.float32),
        scratch_types=[
            pltpu.VMEM((NBUF, CH, D), jnp.float32),
            pltpu.VMEM((NBUF, SB * CH, D), jnp.float32),
            pltpu.SemaphoreType.DMA,
            pltpu.SemaphoreType.DMA,
            pltpu.SemaphoreType.DMA,
            pltpu.SemaphoreType.DMA,
            pltpu.SemaphoreType.DMA,
        ],
    )
    def sc_k(x_hbm, w_hbm, out_hbm, wbuf, xbuf, semw, semx, semo0, semo1, semo2):
        cid = lax.axis_index("c")
        sid = lax.axis_index("s")
        wid = sid * NC + cid
        semo = (semo0, semo1, semo2)

        def start_loads(t, p):
            s0 = wid * per_w + t * CH
            hw = pltpu.async_copy(w_hbm.at[pl.ds(s0, CH)], wbuf.at[p], semw)
            hx = []
            for b in range(SB):
                hx.append(
                    pltpu.async_copy(
                        x_hbm.at[b, pl.ds(s0, CH)],
                        xbuf.at[p, pl.ds(b * CH, CH)],
                        semx,
                    )
                )
            return hw, hx

        def start_stores(t, p):
            s0 = wid * per_w + t * CH
            hs = []
            for b in range(SB):
                hs.append(
                    pltpu.async_copy(
                        xbuf.at[p, pl.ds(b * CH, CH)],
                        out_hbm.at[b, pl.ds(s0, CH)],
                        semo[p],
                    )
                )
            return hs

        loads = [None] * NBUF
        stores = [None] * NBUF
        loads[0] = start_loads(0, 0)
        loads[1] = start_loads(1, 1)
        for t in range(n_chunks):
            p = t % NBUF
            if t + 2 < n_chunks:
                q = (t + 2) % NBUF
                if stores[q] is not None:
                    for h in stores[q]:
                        h.wait()
                    stores[q] = None
                loads[q] = start_loads(t + 2, q)
            hw, hx = loads[p]
            hw.wait()
            for h in hx:
                h.wait()

            @plsc.parallel_loop(0, CH * NV, 1, unroll=8)
            def body(i):
                r = i // NV
                c = (i - r * NV) * 16
                wv = wbuf[p, r, pl.ds(c, 16)]
                for b in range(SB):
                    xbuf[p, b * CH + r, pl.ds(c, 16)] = (
                        xbuf[p, b * CH + r, pl.ds(c, 16)] + wv
                    )

            stores[p] = start_stores(t, p)
        for hs in stores:
            if hs is not None:
                for h in hs:
                    h.wait()

    return sc_k(inputs, weight)


def kernel(inputs, weight):
    sc_out = _sc_part(inputs, weight)
    tc_out = _tc_part(inputs, weight)
    return jnp.concatenate([sc_out, tc_out], axis=0)


# speedup vs baseline: 2.3743x; 2.3743x over previous
import jax
import jax.numpy as jnp
from jax.experimental import pallas as pl

BLOCK_S = 512


def _add_kernel(x_ref, w_ref, o_ref):
    o_ref[...] = x_ref[...] + w_ref[...]


def kernel(inputs, weight):
    batch, seq_len, dim = inputs.shape
    w = weight[:seq_len]
    grid = (seq_len // BLOCK_S,)
    return pl.pallas_call(
        _add_kernel,
        grid=grid,
        in_specs=[
            pl.BlockSpec((batch, BLOCK_S, dim), lambda s: (0, s, 0)),
            pl.BlockSpec((BLOCK_S, dim), lambda s: (s, 0)),
        ],
        out_specs=pl.BlockSpec((batch, BLOCK_S, dim), lambda s: (0, s, 0)),
        out_shape=jax.ShapeDtypeStruct(inputs.shape, inputs.dtype),
    )(inputs, w)
